# baseline (device time: 34145 ns/iter reference)
import jax
import jax.numpy as jnp
from jax import lax
from jax.experimental import pallas as pl
from jax.experimental.pallas import tpu as pltpu

T = 1024
D = 2048
V_SHARD = 16384
NDEV = 16
NSUB = 8
SUBV = V_SHARD // NSUB
VB = 512
NBLK = SUBV // VB


def _coords(p):
    return (p // 8, (p % 8) // 4, p % 4)


def _flat_ids():
    my_x = lax.axis_index("x")
    my_y = lax.axis_index("y")
    my_z = lax.axis_index("z")
    k = my_y * 4 + my_z
    return my_x, k, my_x * 8 + k


def _compute_partials(x, W, labels2d):

    def body(x_ref, w_ref, lab_ref, s_ref, ll_ref,
             xb_ref, wbuf_ref, copy_sems):
        my_x, k, _ = _flat_ids()
        col_base = k * SUBV

        def w_copy(b):
            return pltpu.make_async_copy(
                w_ref.at[:, pl.ds(col_base + b * VB, VB)],
                wbuf_ref.at[b % 3],
                copy_sems.at[b % 3],
            )

        w_copy(0).start()
        w_copy(1).start()
        xb_ref[...] = x_ref[...].astype(jnp.bfloat16)

        lab = lab_ref[...]
        s_acc = jnp.zeros((T, 1), jnp.float32)
        ll_acc = jnp.zeros((T, 1), jnp.float32)

        w_copy(0).wait()
        wb_cur = wbuf_ref[0].astype(jnp.bfloat16)
        for b in range(NBLK):
            if b + 2 < NBLK:
                w_copy(b + 2).start()
            if b + 1 < NBLK:
                w_copy(b + 1).wait()
                wb_next = wbuf_ref[(b + 1) % 3].astype(jnp.bfloat16)
            logits = jnp.dot(xb_ref[...], wb_cur, preferred_element_type=jnp.float32)
            s_acc = s_acc + jnp.sum(jnp.exp(logits), axis=1, keepdims=True)
            gcol0 = my_x * V_SHARD + col_base + b * VB
            cols = gcol0 + lax.broadcasted_iota(jnp.int32, (T, VB), 1)
            ll_acc = ll_acc + jnp.sum(
                jnp.where(cols == lab, logits, 0.0), axis=1, keepdims=True
            )
            if b + 1 < NBLK:
                wb_cur = wb_next

        s_ref[...] = s_acc
        ll_ref[...] = ll_acc

    return pl.pallas_call(
        body,
        in_specs=[
            pl.BlockSpec(memory_space=pltpu.VMEM),
            pl.BlockSpec(memory_space=pl.ANY),
            pl.BlockSpec(memory_space=pltpu.VMEM),
        ],
        out_specs=(
            pl.BlockSpec(memory_space=pltpu.VMEM),
            pl.BlockSpec(memory_space=pltpu.VMEM),
        ),
        out_shape=(
            jax.ShapeDtypeStruct((T, 1), jnp.float32),
            jax.ShapeDtypeStruct((T, 1), jnp.float32),
        ),
        scratch_shapes=[
            pltpu.VMEM((T, D), jnp.bfloat16),
            pltpu.VMEM((3, D, VB), jnp.float32),
            pltpu.SemaphoreType.DMA((3,)),
        ],
    )(x, W, labels2d)


def _all_reduce_combine(s8, ll8):

    def body(s_ref, ll_ref, out_ref, acc_ref, comm_ref, send_sems, recv_sems):
        _, _, my_flat = _flat_ids()

        acc_ref[0:8, :] = s_ref[...]
        acc_ref[8:16, :] = ll_ref[...]

        pltpu.make_async_copy(
            acc_ref, comm_ref.at[my_flat], recv_sems.at[my_flat]
        ).start()

        for p in range(NDEV):
            @pl.when(my_flat != p)
            def _(p=p):
                pltpu.make_async_remote_copy(
                    src_ref=acc_ref,
                    dst_ref=comm_ref.at[my_flat],
                    send_sem=send_sems.at[p],
                    recv_sem=recv_sems.at[my_flat],
                    device_id=_coords(p),
                    device_id_type=pl.DeviceIdType.MESH,
                ).start()

        for s in range(NDEV):
            pltpu.make_async_copy(
                acc_ref, comm_ref.at[s], recv_sems.at[s]
            ).wait()

        for p in range(NDEV):
            @pl.when(my_flat != p)
            def _(p=p):
                pltpu.make_async_remote_copy(
                    src_ref=acc_ref,
                    dst_ref=comm_ref.at[my_flat],
                    send_sem=send_sems.at[p],
                    recv_sem=recv_sems.at[my_flat],
                    device_id=_coords(p),
                    device_id_type=pl.DeviceIdType.MESH,
                ).wait_send()

        tot = jnp.sum(comm_ref[...], axis=0)
        out_ref[...] = jnp.log(tot[0:8, :]) - tot[8:16, :]

    return pl.pallas_call(
        body,
        in_specs=[
            pl.BlockSpec(memory_space=pltpu.VMEM),
            pl.BlockSpec(memory_space=pltpu.VMEM),
        ],
        out_specs=pl.BlockSpec(memory_space=pltpu.VMEM),
        out_shape=jax.ShapeDtypeStruct((8, 128), jnp.float32),
        scratch_shapes=[
            pltpu.VMEM((16, 128), jnp.float32),
            pltpu.VMEM((NDEV, 16, 128), jnp.float32),
            pltpu.SemaphoreType.DMA((NDEV,)),
            pltpu.SemaphoreType.DMA((NDEV,)),
        ],
    )(s8, ll8)


def kernel(x, W, labels):
    labels2d = labels.reshape(T, 1)
    s, ll = _compute_partials(x, W, labels2d)
    out8 = _all_reduce_combine(s.reshape(8, 128), ll.reshape(8, 128))
    return out8.reshape(T)


# device time: 33380 ns/iter; 1.0229x vs baseline; 1.0229x over previous
import jax
import jax.numpy as jnp
from jax import lax
from jax.experimental import pallas as pl
from jax.experimental.pallas import tpu as pltpu

T = 1024
D = 2048
V_SHARD = 16384
NDEV = 16
NSUB = 8
SUBV = V_SHARD // NSUB
VB = 512
NBLK = SUBV // VB


def _coords(p):
    return (p // 8, (p % 8) // 4, p % 4)


def _flat_ids():
    my_x = lax.axis_index("x")
    my_y = lax.axis_index("y")
    my_z = lax.axis_index("z")
    k = my_y * 4 + my_z
    return my_x, k, my_x * 8 + k


def _compute_partials(x, W, labels2d):

    def body(x_ref, w_ref, lab_ref, s_ref, ll_ref,
             xb_ref, wbuf_ref, copy_sems):
        my_x, k, _ = _flat_ids()
        col_base = k * SUBV

        xb_ref[...] = x_ref[...].astype(jnp.bfloat16)

        def w_copy(b, slot):
            return pltpu.make_async_copy(
                w_ref.at[:, pl.ds(col_base + b * VB, VB)],
                wbuf_ref.at[slot],
                copy_sems.at[slot],
            )

        w_copy(0, 0).start()

        lab = lab_ref[...]
        s_acc = jnp.zeros((T, 1), jnp.float32)
        ll_acc = jnp.zeros((T, 1), jnp.float32)
        for b in range(NBLK):
            slot = b % 2
            if b + 1 < NBLK:
                w_copy(b + 1, 1 - slot).start()
            w_copy(b, slot).wait()
            wb = wbuf_ref[slot].astype(jnp.bfloat16)
            logits = jnp.dot(xb_ref[...], wb, preferred_element_type=jnp.float32)
            s_acc = s_acc + jnp.sum(jnp.exp(logits), axis=1, keepdims=True)
            gcol0 = my_x * V_SHARD + col_base + b * VB
            cols = gcol0 + lax.broadcasted_iota(jnp.int32, (T, VB), 1)
            ll_acc = ll_acc + jnp.sum(
                jnp.where(cols == lab, logits, 0.0), axis=1, keepdims=True
            )

        s_ref[...] = s_acc
        ll_ref[...] = ll_acc

    return pl.pallas_call(
        body,
        in_specs=[
            pl.BlockSpec(memory_space=pltpu.VMEM),
            pl.BlockSpec(memory_space=pl.ANY),
            pl.BlockSpec(memory_space=pltpu.VMEM),
        ],
        out_specs=(
            pl.BlockSpec(memory_space=pltpu.VMEM),
            pl.BlockSpec(memory_space=pltpu.VMEM),
        ),
        out_shape=(
            jax.ShapeDtypeStruct((T, 1), jnp.float32),
            jax.ShapeDtypeStruct((T, 1), jnp.float32),
        ),
        scratch_shapes=[
            pltpu.VMEM((T, D), jnp.bfloat16),
            pltpu.VMEM((2, D, VB), jnp.float32),
            pltpu.SemaphoreType.DMA((2,)),
        ],
    )(x, W, labels2d)


def _all_reduce_combine(s8, ll8):

    def body(s_ref, ll_ref, out_ref, acc_ref, comm_ref, send_sems, recv_sems):
        _, _, my_flat = _flat_ids()

        acc_ref[0:8, :] = s_ref[...]
        acc_ref[8:16, :] = ll_ref[...]

        pltpu.make_async_copy(
            acc_ref, comm_ref.at[my_flat], recv_sems.at[my_flat]
        ).start()

        for p in range(NDEV):
            @pl.when(my_flat != p)
            def _(p=p):
                pltpu.make_async_remote_copy(
                    src_ref=acc_ref,
                    dst_ref=comm_ref.at[my_flat],
                    send_sem=send_sems.at[p],
                    recv_sem=recv_sems.at[my_flat],
                    device_id=_coords(p),
                    device_id_type=pl.DeviceIdType.MESH,
                ).start()

        for s in range(NDEV):
            pltpu.make_async_copy(
                acc_ref, comm_ref.at[s], recv_sems.at[s]
            ).wait()

        for p in range(NDEV):
            @pl.when(my_flat != p)
            def _(p=p):
                pltpu.make_async_remote_copy(
                    src_ref=acc_ref,
                    dst_ref=comm_ref.at[my_flat],
                    send_sem=send_sems.at[p],
                    recv_sem=recv_sems.at[my_flat],
                    device_id=_coords(p),
                    device_id_type=pl.DeviceIdType.MESH,
                ).wait_send()

        tot = jnp.sum(comm_ref[...], axis=0)
        out_ref[...] = jnp.log(tot[0:8, :]) - tot[8:16, :]

    return pl.pallas_call(
        body,
        in_specs=[
            pl.BlockSpec(memory_space=pltpu.VMEM),
            pl.BlockSpec(memory_space=pltpu.VMEM),
        ],
        out_specs=pl.BlockSpec(memory_space=pltpu.VMEM),
        out_shape=jax.ShapeDtypeStruct((8, 128), jnp.float32),
        scratch_shapes=[
            pltpu.VMEM((16, 128), jnp.float32),
            pltpu.VMEM((NDEV, 16, 128), jnp.float32),
            pltpu.SemaphoreType.DMA((NDEV,)),
            pltpu.SemaphoreType.DMA((NDEV,)),
        ],
    )(s8, ll8)


def kernel(x, W, labels):
    labels2d = labels.reshape(T, 1)
    s, ll = _compute_partials(x, W, labels2d)
    out8 = _all_reduce_combine(s.reshape(8, 128), ll.reshape(8, 128))
    return out8.reshape(T)
